# baseline (speedup 1.0000x reference)
import functools

import jax
import jax.numpy as jnp
from jax import lax
from jax.experimental import pallas as pl
from jax.experimental.pallas import tpu as pltpu
from jax.experimental.pallas import tpu_sc as plsc

N = 10000
D = 128
E = 320000
G = 256
NC = 2
NS = 16
K = 64
CH0 = 228
CH1 = 96
E_PAD = NS * (CH0 + CH1) * K
NPAD = N + 8
RPT = 1000
RB = 1000
NRB = N // RB
EPS = 1e-5


_SC_MESH = plsc.VectorSubcoreMesh(core_axis_name="c", subcore_axis_name="s")


def _edge_pipeline(h_hbm, src_hbm, dst_hbm, n_chunks, s,
                   src_i, dst_i, rows_v, deg_l, agg_sh, gsem, ssem, isem):
    ones16 = jnp.ones((16,), jnp.float32)
    nj = n_chunks // 12

    def idx_wait(t):
        pltpu.make_async_copy(src_hbm.at[s, 0], src_i[t], isem).wait()
        pltpu.make_async_copy(dst_hbm.at[s, 0], dst_i[t], isem).wait()

    def gather_wait(t, r):
        pltpu.make_async_copy(h_hbm.at[src_i[t]], rows_v[r], gsem[r]).wait()

    def scatter_wait(t, r):
        pltpu.make_async_copy(rows_v[r], agg_sh.at[dst_i[t]], ssem[r]).wait()

    pltpu.sync_copy(src_hbm.at[s, 0], src_i[0])
    pltpu.sync_copy(dst_hbm.at[s, 0], dst_i[0])
    pltpu.sync_copy(src_hbm.at[s, 1], src_i[1])
    pltpu.sync_copy(dst_hbm.at[s, 1], dst_i[1])
    pltpu.async_copy(src_hbm.at[s, 2], src_i[2], isem)
    pltpu.async_copy(dst_hbm.at[s, 2], dst_i[2], isem)
    pltpu.async_copy(h_hbm.at[src_i[0]], rows_v[0], gsem[0])
    pltpu.async_copy(h_hbm.at[src_i[1]], rows_v[1], gsem[1])

    def body(jj, carry):
        for u in range(12):
            j = jj * 12 + u
            r = u % 4
            r2 = (u + 2) % 4
            t = u % 6
            t2 = (u + 2) % 6
            t3 = (u + 3) % 6

            if u >= 2:
                scatter_wait((u - 2) % 6, r2)
            else:
                @pl.when(jj > 0)
                def _(u=u, r2=r2):
                    scatter_wait((u - 2) % 6, r2)

            if u < 9:
                pltpu.async_copy(src_hbm.at[s, j + 3], src_i[t3], isem)
                pltpu.async_copy(dst_hbm.at[s, j + 3], dst_i[t3], isem)
            else:
                @pl.when(jj < nj - 1)
                def _(j=j, t3=t3):
                    pltpu.async_copy(src_hbm.at[s, j + 3], src_i[t3], isem)
                    pltpu.async_copy(dst_hbm.at[s, j + 3], dst_i[t3], isem)

            for k in range(K // 16):
                d16 = dst_i[t][pl.ds(k * 16, 16)]
                plsc.addupdate_scatter(deg_l, [d16], ones16)

            gather_wait(t, r)
            pltpu.async_copy(rows_v[r], agg_sh.at[dst_i[t]], ssem[r],
                             add=True)

            if u < 10:
                idx_wait(t2)
                pltpu.async_copy(h_hbm.at[src_i[t2]], rows_v[r2], gsem[r2])
            else:
                @pl.when(jj < nj - 1)
                def _(t2=t2, r2=r2):
                    idx_wait(t2)
                    pltpu.async_copy(h_hbm.at[src_i[t2]], rows_v[r2],
                                     gsem[r2])
        return carry

    lax.fori_loop(0, nj, body, 0)
    scatter_wait(4, 2)
    scatter_wait(5, 3)


@functools.partial(
    pl.kernel,
    out_type=[
        jax.ShapeDtypeStruct((NC, N, D), jnp.float32),
        jax.ShapeDtypeStruct((NC * NS, NPAD), jnp.float32),
    ],
    mesh=_SC_MESH,
    scratch_types=(
        [pltpu.VMEM((K,), jnp.int32)] * 6
        + [pltpu.VMEM((K,), jnp.int32)] * 6
        + [pltpu.VMEM((K, D), jnp.float32)] * 4
        + [
            pltpu.VMEM((NPAD,), jnp.float32),
            pltpu.VMEM_SHARED((NPAD, D), jnp.float32),
        ]
        + [pltpu.SemaphoreType.DMA] * 9
    ),
    compiler_params=pltpu.CompilerParams(needs_layout_passes=False),
)
def _sc_gather_scatter(h_hbm, src0_hbm, dst0_hbm, src1_hbm, dst1_hbm,
                       zrows_hbm, zdeg_hbm,
                       agg_out, deg_out,
                       si0, si1, si2, si3, si4, si5,
                       di0, di1, di2, di3, di4, di5,
                       rv0, rv1, rv2, rv3, deg_l, agg_sh,
                       gs0, gs1, gs2, gs3, ss0, ss1, ss2, ss3, isem):
    c = lax.axis_index("c")
    s = lax.axis_index("s")
    src_i = (si0, si1, si2, si3, si4, si5)
    dst_i = (di0, di1, di2, di3, di4, di5)
    rows_v = (rv0, rv1, rv2, rv3)
    gsem = (gs0, gs1, gs2, gs3)
    ssem = (ss0, ss1, ss2, ss3)

    @pl.when(s < N // RPT)
    def _():
        pltpu.sync_copy(zrows_hbm, agg_sh.at[pl.ds(s * RPT, RPT)])

    pltpu.sync_copy(zdeg_hbm, deg_l)
    plsc.subcore_barrier()

    @pl.when(c == 0)
    def _():
        _edge_pipeline(h_hbm, src0_hbm, dst0_hbm, CH0, s,
                       src_i, dst_i, rows_v, deg_l, agg_sh, gsem, ssem, isem)

    @pl.when(c == 1)
    def _():
        _edge_pipeline(h_hbm, src1_hbm, dst1_hbm, CH1, s,
                       src_i, dst_i, rows_v, deg_l, agg_sh, gsem, ssem, isem)

    plsc.subcore_barrier()

    @pl.when(s < N // RPT)
    def _():
        pltpu.sync_copy(agg_sh.at[pl.ds(s * RPT, RPT)],
                        agg_out.at[c, pl.ds(s * RPT, RPT)])

    pltpu.sync_copy(deg_l, deg_out.at[c * NS + s])


def _proj_body(x_ref, w_ref, b_ref, o_ref):
    o_ref[...] = jnp.dot(x_ref[...], w_ref[...],
                         preferred_element_type=jnp.float32) + b_ref[...]


def _sage_block(agg_ref, deg_ref, h_ref, wl_ref, wr_ref, bl_ref, g_ref, be_ref):
    a = agg_ref[0] + agg_ref[1]
    deg = jnp.maximum(jnp.sum(deg_ref[0], axis=0), 1.0)
    a = a / deg[:, None]
    t = (jnp.dot(a, wl_ref[...], preferred_element_type=jnp.float32)
         + jnp.dot(h_ref[...], wr_ref[...], preferred_element_type=jnp.float32)
         + bl_ref[...])
    t = jnp.where(t > 0, t, jnp.exp(jnp.minimum(t, 0.0)) - 1.0)
    mu = jnp.mean(t, axis=-1, keepdims=True)
    var = jnp.mean((t - mu) ** 2, axis=-1, keepdims=True)
    return (t - mu) / jnp.sqrt(var + EPS) * g_ref[...] + be_ref[...]


def _combine_body(agg_ref, deg_ref, h_ref, wl_ref, wr_ref, bl_ref, g_ref,
                  be_ref, o_ref):
    o_ref[...] = _sage_block(agg_ref, deg_ref, h_ref, wl_ref, wr_ref, bl_ref,
                             g_ref, be_ref)


def _combine_pool_body(agg_ref, deg_ref, h_ref, wl_ref, wr_ref, bl_ref, g_ref,
                       be_ref, batch_ref, wmt_ref, bmt_ref, o_ref,
                       sum_acc, cnt_acc):
    i = pl.program_id(0)

    @pl.when(i == 0)
    def _():
        sum_acc[...] = jnp.zeros_like(sum_acc)
        cnt_acc[...] = jnp.zeros_like(cnt_acc)

    hh = _sage_block(agg_ref, deg_ref, h_ref, wl_ref, wr_ref, bl_ref, g_ref,
                     be_ref)
    gi = lax.broadcasted_iota(jnp.int32, (G, RB), 0)
    oh = (batch_ref[0, 0, :][None, :] == gi).astype(jnp.float32)
    sum_acc[...] += jnp.dot(oh, hh, preferred_element_type=jnp.float32)
    cnt_acc[...] += jnp.broadcast_to(jnp.sum(oh, axis=1, keepdims=True), (G, D))

    @pl.when(i == NRB - 1)
    def _():
        emb = sum_acc[...] / jnp.maximum(cnt_acc[...], 1.0)
        o_ref[...] = jnp.dot(emb, wmt_ref[...],
                             preferred_element_type=jnp.float32) + bmt_ref[...]


_ROW_SPECS = [
    pl.BlockSpec((2, RB, D), lambda i: (0, i, 0)),
    pl.BlockSpec((1, NC * NS, RB), lambda i: (i, 0, 0)),
    pl.BlockSpec((RB, D), lambda i: (i, 0)),
    pl.BlockSpec((D, D), lambda i: (0, 0)),
    pl.BlockSpec((D, D), lambda i: (0, 0)),
    pl.BlockSpec((1, D), lambda i: (0, 0)),
    pl.BlockSpec((1, D), lambda i: (0, 0)),
    pl.BlockSpec((1, D), lambda i: (0, 0)),
]

_proj = pl.pallas_call(
    _proj_body,
    grid=(NRB,),
    in_specs=[
        pl.BlockSpec((RB, D), lambda i: (i, 0)),
        pl.BlockSpec((D, D), lambda i: (0, 0)),
        pl.BlockSpec((1, D), lambda i: (0, 0)),
    ],
    out_specs=pl.BlockSpec((RB, D), lambda i: (i, 0)),
    out_shape=jax.ShapeDtypeStruct((N, D), jnp.float32),
)

_combine = pl.pallas_call(
    _combine_body,
    grid=(NRB,),
    in_specs=_ROW_SPECS,
    out_specs=pl.BlockSpec((RB, D), lambda i: (i, 0)),
    out_shape=jax.ShapeDtypeStruct((N, D), jnp.float32),
)

_combine_pool = pl.pallas_call(
    _combine_pool_body,
    grid=(NRB,),
    in_specs=_ROW_SPECS + [
        pl.BlockSpec((1, 1, RB), lambda i: (i, 0, 0)),
        pl.BlockSpec((D, D), lambda i: (0, 0)),
        pl.BlockSpec((1, D), lambda i: (0, 0)),
    ],
    out_specs=pl.BlockSpec((G, D), lambda i: (0, 0)),
    out_shape=jax.ShapeDtypeStruct((G, D), jnp.float32),
    scratch_shapes=[
        pltpu.VMEM((G, D), jnp.float32),
        pltpu.VMEM((G, D), jnp.float32),
    ],
)


def kernel(x_operator, edge_index_calledby, batch_operator, W_op, b_op,
           W_l, b_l, W_r, gamma, beta, W_mem, b_mem, W_time, b_time):
    src = edge_index_calledby[0].astype(jnp.int32)
    dst = edge_index_calledby[1].astype(jnp.int32)
    pad = E_PAD - E
    e0 = NS * CH0 * K
    src_p = jnp.concatenate([src, jnp.zeros((pad,), jnp.int32)])
    src0 = src_p[:e0].reshape(NS, CH0, K)
    src1 = src_p[e0:].reshape(NS, CH1, K)
    dst_p = jnp.concatenate([dst, jnp.full((pad,), N, jnp.int32)])
    dst0 = dst_p[:e0].reshape(NS, CH0, K)
    dst1 = dst_p[e0:].reshape(NS, CH1, K)
    batch3 = batch_operator.astype(jnp.int32).reshape(NRB, 1, RB)

    zrows = jnp.zeros((RPT, D), jnp.float32)
    zdeg = jnp.zeros((NPAD,), jnp.float32)

    b_op2 = b_op.reshape(1, D)
    bl2 = b_l.reshape(1, D)
    g2 = gamma.reshape(1, D)
    be2 = beta.reshape(1, D)
    wmt = jnp.zeros((D, D), jnp.float32)
    wmt = wmt.at[:, 0].set(W_mem[:, 0]).at[:, 1].set(W_time[:, 0])
    bmt = jnp.zeros((1, D), jnp.float32)
    bmt = bmt.at[0, 0].set(b_mem[0]).at[0, 1].set(b_time[0])

    h0 = _proj(x_operator, W_op, b_op2)
    aggp1, degp = _sc_gather_scatter(h0, src0, dst0, src1, dst1, zrows, zdeg)
    deg3 = degp[:, :N].reshape(NC * NS, NRB, RB).transpose(1, 0, 2)
    h1 = _combine(aggp1, deg3, h0, W_l, W_r, bl2, g2, be2)
    aggp2, _ = _sc_gather_scatter(h1, src0, dst0, src1, dst1, zrows, zdeg)
    out = _combine_pool(aggp2, deg3, h1, W_l, W_r, bl2, g2, be2,
                        batch3, wmt, bmt)
    return out[:, 0], out[:, 1]


# speedup vs baseline: 1.0958x; 1.0958x over previous
import functools

import jax
import jax.numpy as jnp
from jax import lax
from jax.experimental import pallas as pl
from jax.experimental.pallas import tpu as pltpu
from jax.experimental.pallas import tpu_sc as plsc

N = 10000
D = 128
E = 320000
G = 256
NC = 2
NS = 16
K = 128
CH0 = 114
CH1 = 48
E_PAD = NS * (CH0 + CH1) * K
NPAD = N + 8
RPT = 1000
RB = 1000
NRB = N // RB
EPS = 1e-5


_SC_MESH = plsc.VectorSubcoreMesh(core_axis_name="c", subcore_axis_name="s")


def _edge_pipeline(h_hbm, src_hbm, dst_hbm, n_chunks, s,
                   src_i, dst_i, rows_v, deg_l, agg_sh, gsem, ssem, isem):
    ones16 = jnp.ones((16,), jnp.float32)
    nj = n_chunks // 6

    def gather_wait(t, b):
        pltpu.make_async_copy(h_hbm.at[src_i[t]], rows_v[b], gsem[b]).wait()

    def scatter_wait(t, b):
        pltpu.make_async_copy(rows_v[b], agg_sh.at[dst_i[t]], ssem[b]).wait()

    pltpu.sync_copy(src_hbm.at[s, 0], src_i[0])
    pltpu.sync_copy(dst_hbm.at[s, 0], dst_i[0])
    pltpu.async_copy(h_hbm.at[src_i[0]], rows_v[0], gsem[0])

    def body(jj, carry):
        s_prev = None
        g_cur = None
        for u in range(6):
            j = jj * 6 + u
            b, bp = u % 2, 1 - u % 2
            t, tn = u % 3, (u + 1) % 3

            if u < 5:
                ia = pltpu.async_copy(src_hbm.at[s, j + 1], src_i[tn], isem)
                ib = pltpu.async_copy(dst_hbm.at[s, j + 1], dst_i[tn], isem)

            for k in range(K // 16):
                d16 = dst_i[t][pl.ds(k * 16, 16)]
                plsc.addupdate_scatter(deg_l, [d16], ones16)

            if g_cur is None:
                gather_wait(t, b)
            else:
                g_cur.wait()
            s_cur = pltpu.async_copy(rows_v[b], agg_sh.at[dst_i[t]], ssem[b],
                                     add=True)

            if s_prev is None:
                @pl.when(jj > 0)
                def _():
                    scatter_wait(2, 1)
            else:
                s_prev.wait()
            s_prev = s_cur

            if u < 5:
                ia.wait()
                ib.wait()
                g_cur = pltpu.async_copy(h_hbm.at[src_i[tn]], rows_v[bp],
                                         gsem[bp])
            else:
                @pl.when(jj < nj - 1)
                def _():
                    pltpu.async_copy(src_hbm.at[s, j + 1], src_i[tn],
                                     isem).wait()
                    pltpu.async_copy(dst_hbm.at[s, j + 1], dst_i[tn],
                                     isem).wait()
                    pltpu.async_copy(h_hbm.at[src_i[tn]], rows_v[bp], gsem[bp])
                g_cur = None
        return carry

    lax.fori_loop(0, nj, body, 0)
    scatter_wait(2, 1)


@functools.partial(
    pl.kernel,
    out_type=[
        jax.ShapeDtypeStruct((NC, N, D), jnp.float32),
        jax.ShapeDtypeStruct((NC * NS, NPAD), jnp.float32),
    ],
    mesh=_SC_MESH,
    scratch_types=[
        pltpu.VMEM((K,), jnp.int32),
        pltpu.VMEM((K,), jnp.int32),
        pltpu.VMEM((K,), jnp.int32),
        pltpu.VMEM((K,), jnp.int32),
        pltpu.VMEM((K,), jnp.int32),
        pltpu.VMEM((K,), jnp.int32),
        pltpu.VMEM((K, D), jnp.float32),
        pltpu.VMEM((K, D), jnp.float32),
        pltpu.VMEM((NPAD,), jnp.float32),
        pltpu.VMEM_SHARED((NPAD, D), jnp.float32),
        pltpu.SemaphoreType.DMA,
        pltpu.SemaphoreType.DMA,
        pltpu.SemaphoreType.DMA,
        pltpu.SemaphoreType.DMA,
        pltpu.SemaphoreType.DMA,
    ],
    compiler_params=pltpu.CompilerParams(needs_layout_passes=False),
)
def _sc_gather_scatter(h_hbm, src0_hbm, dst0_hbm, src1_hbm, dst1_hbm,
                       zrows_hbm, zdeg_hbm,
                       agg_out, deg_out,
                       src_i0, src_i1, src_i2, dst_i0, dst_i1, dst_i2,
                       rows_v0, rows_v1, deg_l, agg_sh,
                       gsem0, gsem1, ssem0, ssem1, isem):
    c = lax.axis_index("c")
    s = lax.axis_index("s")
    src_i = (src_i0, src_i1, src_i2)
    dst_i = (dst_i0, dst_i1, dst_i2)
    rows_v = (rows_v0, rows_v1)
    gsem = (gsem0, gsem1)
    ssem = (ssem0, ssem1)

    @pl.when(s < N // RPT)
    def _():
        pltpu.sync_copy(zrows_hbm, agg_sh.at[pl.ds(s * RPT, RPT)])

    pltpu.sync_copy(zdeg_hbm, deg_l)
    plsc.subcore_barrier()

    @pl.when(c == 0)
    def _():
        _edge_pipeline(h_hbm, src0_hbm, dst0_hbm, CH0, s,
                       src_i, dst_i, rows_v, deg_l, agg_sh, gsem, ssem, isem)

    @pl.when(c == 1)
    def _():
        _edge_pipeline(h_hbm, src1_hbm, dst1_hbm, CH1, s,
                       src_i, dst_i, rows_v, deg_l, agg_sh, gsem, ssem, isem)

    plsc.subcore_barrier()

    @pl.when(s < N // RPT)
    def _():
        pltpu.sync_copy(agg_sh.at[pl.ds(s * RPT, RPT)],
                        agg_out.at[c, pl.ds(s * RPT, RPT)])

    pltpu.sync_copy(deg_l, deg_out.at[c * NS + s])


def _proj_body(x_ref, w_ref, b_ref, o_ref):
    o_ref[...] = jnp.dot(x_ref[...], w_ref[...],
                         preferred_element_type=jnp.float32) + b_ref[...]


def _sage_block(agg_ref, deg_ref, h_ref, wl_ref, wr_ref, bl_ref, g_ref, be_ref):
    a = agg_ref[0] + agg_ref[1]
    deg = jnp.maximum(jnp.sum(deg_ref[0], axis=0), 1.0)
    a = a / deg[:, None]
    t = (jnp.dot(a, wl_ref[...], preferred_element_type=jnp.float32)
         + jnp.dot(h_ref[...], wr_ref[...], preferred_element_type=jnp.float32)
         + bl_ref[...])
    t = jnp.where(t > 0, t, jnp.exp(jnp.minimum(t, 0.0)) - 1.0)
    mu = jnp.mean(t, axis=-1, keepdims=True)
    var = jnp.mean((t - mu) ** 2, axis=-1, keepdims=True)
    return (t - mu) / jnp.sqrt(var + EPS) * g_ref[...] + be_ref[...]


def _combine_body(agg_ref, deg_ref, h_ref, wl_ref, wr_ref, bl_ref, g_ref,
                  be_ref, o_ref):
    o_ref[...] = _sage_block(agg_ref, deg_ref, h_ref, wl_ref, wr_ref, bl_ref,
                             g_ref, be_ref)


def _combine_pool_body(agg_ref, deg_ref, h_ref, wl_ref, wr_ref, bl_ref, g_ref,
                       be_ref, batch_ref, wmt_ref, bmt_ref, o_ref,
                       sum_acc, cnt_acc):
    i = pl.program_id(0)

    @pl.when(i == 0)
    def _():
        sum_acc[...] = jnp.zeros_like(sum_acc)
        cnt_acc[...] = jnp.zeros_like(cnt_acc)

    hh = _sage_block(agg_ref, deg_ref, h_ref, wl_ref, wr_ref, bl_ref, g_ref,
                     be_ref)
    gi = lax.broadcasted_iota(jnp.int32, (G, RB), 0)
    oh = (batch_ref[0, 0, :][None, :] == gi).astype(jnp.float32)
    sum_acc[...] += jnp.dot(oh, hh, preferred_element_type=jnp.float32)
    cnt_acc[...] += jnp.broadcast_to(jnp.sum(oh, axis=1, keepdims=True), (G, D))

    @pl.when(i == NRB - 1)
    def _():
        emb = sum_acc[...] / jnp.maximum(cnt_acc[...], 1.0)
        o_ref[...] = jnp.dot(emb, wmt_ref[...],
                             preferred_element_type=jnp.float32) + bmt_ref[...]


_ROW_SPECS = [
    pl.BlockSpec((2, RB, D), lambda i: (0, i, 0)),
    pl.BlockSpec((1, NC * NS, RB), lambda i: (i, 0, 0)),
    pl.BlockSpec((RB, D), lambda i: (i, 0)),
    pl.BlockSpec((D, D), lambda i: (0, 0)),
    pl.BlockSpec((D, D), lambda i: (0, 0)),
    pl.BlockSpec((1, D), lambda i: (0, 0)),
    pl.BlockSpec((1, D), lambda i: (0, 0)),
    pl.BlockSpec((1, D), lambda i: (0, 0)),
]

_proj = pl.pallas_call(
    _proj_body,
    grid=(NRB,),
    in_specs=[
        pl.BlockSpec((RB, D), lambda i: (i, 0)),
        pl.BlockSpec((D, D), lambda i: (0, 0)),
        pl.BlockSpec((1, D), lambda i: (0, 0)),
    ],
    out_specs=pl.BlockSpec((RB, D), lambda i: (i, 0)),
    out_shape=jax.ShapeDtypeStruct((N, D), jnp.float32),
)

_combine = pl.pallas_call(
    _combine_body,
    grid=(NRB,),
    in_specs=_ROW_SPECS,
    out_specs=pl.BlockSpec((RB, D), lambda i: (i, 0)),
    out_shape=jax.ShapeDtypeStruct((N, D), jnp.float32),
)

_combine_pool = pl.pallas_call(
    _combine_pool_body,
    grid=(NRB,),
    in_specs=_ROW_SPECS + [
        pl.BlockSpec((1, 1, RB), lambda i: (i, 0, 0)),
        pl.BlockSpec((D, D), lambda i: (0, 0)),
        pl.BlockSpec((1, D), lambda i: (0, 0)),
    ],
    out_specs=pl.BlockSpec((G, D), lambda i: (0, 0)),
    out_shape=jax.ShapeDtypeStruct((G, D), jnp.float32),
    scratch_shapes=[
        pltpu.VMEM((G, D), jnp.float32),
        pltpu.VMEM((G, D), jnp.float32),
    ],
)


def kernel(x_operator, edge_index_calledby, batch_operator, W_op, b_op,
           W_l, b_l, W_r, gamma, beta, W_mem, b_mem, W_time, b_time):
    src = edge_index_calledby[0].astype(jnp.int32)
    dst = edge_index_calledby[1].astype(jnp.int32)
    pad = E_PAD - E
    e0 = NS * CH0 * K
    src_p = jnp.concatenate([src, jnp.zeros((pad,), jnp.int32)])
    src0 = src_p[:e0].reshape(NS, CH0, K)
    src1 = src_p[e0:].reshape(NS, CH1, K)
    dst_p = jnp.concatenate([dst, jnp.full((pad,), N, jnp.int32)])
    dst0 = dst_p[:e0].reshape(NS, CH0, K)
    dst1 = dst_p[e0:].reshape(NS, CH1, K)
    batch3 = batch_operator.astype(jnp.int32).reshape(NRB, 1, RB)

    zrows = jnp.zeros((RPT, D), jnp.float32)
    zdeg = jnp.zeros((NPAD,), jnp.float32)

    b_op2 = b_op.reshape(1, D)
    bl2 = b_l.reshape(1, D)
    g2 = gamma.reshape(1, D)
    be2 = beta.reshape(1, D)
    wmt = jnp.zeros((D, D), jnp.float32)
    wmt = wmt.at[:, 0].set(W_mem[:, 0]).at[:, 1].set(W_time[:, 0])
    bmt = jnp.zeros((1, D), jnp.float32)
    bmt = bmt.at[0, 0].set(b_mem[0]).at[0, 1].set(b_time[0])

    h0 = _proj(x_operator, W_op, b_op2)
    aggp1, degp = _sc_gather_scatter(h0, src0, dst0, src1, dst1, zrows, zdeg)
    deg3 = degp[:, :N].reshape(NC * NS, NRB, RB).transpose(1, 0, 2)
    h1 = _combine(aggp1, deg3, h0, W_l, W_r, bl2, g2, be2)
    aggp2, _ = _sc_gather_scatter(h1, src0, dst0, src1, dst1, zrows, zdeg)
    out = _combine_pool(aggp2, deg3, h1, W_l, W_r, bl2, g2, be2,
                        batch3, wmt, bmt)
    return out[:, 0], out[:, 1]


# speedup vs baseline: 2.4720x; 2.2559x over previous
import functools

import jax
import jax.numpy as jnp
from jax import lax
from jax.experimental import pallas as pl
from jax.experimental.pallas import tpu as pltpu
from jax.experimental.pallas import tpu_sc as plsc

N = 10000
D = 128
E = 320000
G = 256
NC = 2
NS = 16
K = 112
CH0 = 126
CH1 = 54
E_PAD = NS * (CH0 + CH1) * K
NPAD = N + 8
RPT = 1000
RB = 1000
NRB = N // RB
EPS = 1e-5


_SC_MESH = plsc.VectorSubcoreMesh(core_axis_name="c", subcore_axis_name="s")


def _edge_pipeline(h_hbm, src_hbm, dst_hbm, n_chunks, s,
                   src_i, dst_i, rows_v, deg_l, agg_sh, gsem, ssem, isem):
    ones16 = jnp.ones((16,), jnp.float32)
    nj = n_chunks // 6

    def gather_wait(t, b):
        pltpu.make_async_copy(h_hbm.at[src_i[t]], rows_v[b], gsem[b]).wait()

    def scatter_wait(t, b):
        pltpu.make_async_copy(rows_v[b], agg_sh.at[dst_i[t]], ssem[b]).wait()

    pltpu.sync_copy(src_hbm.at[s, 0], src_i[0])
    pltpu.sync_copy(dst_hbm.at[s, 0], dst_i[0])
    pltpu.async_copy(h_hbm.at[src_i[0]], rows_v[0], gsem[0])

    def body(jj, carry):
        s_prev = None
        g_cur = None
        for u in range(6):
            j = jj * 6 + u
            b, bp = u % 2, 1 - u % 2
            t, tn = u % 3, (u + 1) % 3

            if u < 5:
                ia = pltpu.async_copy(src_hbm.at[s, j + 1], src_i[tn], isem)
                ib = pltpu.async_copy(dst_hbm.at[s, j + 1], dst_i[tn], isem)

            for k in range(K // 16):
                d16 = dst_i[t][pl.ds(k * 16, 16)]
                plsc.addupdate_scatter(deg_l, [d16], ones16)

            if g_cur is None:
                gather_wait(t, b)
            else:
                g_cur.wait()
            s_cur = pltpu.async_copy(rows_v[b], agg_sh.at[dst_i[t]], ssem[b],
                                     add=True)

            if s_prev is None:
                @pl.when(jj > 0)
                def _():
                    scatter_wait(2, 1)
            else:
                s_prev.wait()
            s_prev = s_cur

            if u < 5:
                ia.wait()
                ib.wait()
                g_cur = pltpu.async_copy(h_hbm.at[src_i[tn]], rows_v[bp],
                                         gsem[bp])
            else:
                @pl.when(jj < nj - 1)
                def _():
                    pltpu.async_copy(src_hbm.at[s, j + 1], src_i[tn],
                                     isem).wait()
                    pltpu.async_copy(dst_hbm.at[s, j + 1], dst_i[tn],
                                     isem).wait()
                    pltpu.async_copy(h_hbm.at[src_i[tn]], rows_v[bp], gsem[bp])
                g_cur = None
        return carry

    lax.fori_loop(0, nj, body, 0)
    scatter_wait(2, 1)


@functools.partial(
    pl.kernel,
    out_type=[
        jax.ShapeDtypeStruct((NC, N, D), jnp.float32),
        jax.ShapeDtypeStruct((NC * NS, NPAD), jnp.float32),
    ],
    mesh=_SC_MESH,
    scratch_types=[
        pltpu.VMEM((K,), jnp.int32),
        pltpu.VMEM((K,), jnp.int32),
        pltpu.VMEM((K,), jnp.int32),
        pltpu.VMEM((K,), jnp.int32),
        pltpu.VMEM((K,), jnp.int32),
        pltpu.VMEM((K,), jnp.int32),
        pltpu.VMEM((K, D), jnp.float32),
        pltpu.VMEM((K, D), jnp.float32),
        pltpu.VMEM((NPAD,), jnp.float32),
        pltpu.VMEM_SHARED((NPAD, D), jnp.float32),
        pltpu.SemaphoreType.DMA,
        pltpu.SemaphoreType.DMA,
        pltpu.SemaphoreType.DMA,
        pltpu.SemaphoreType.DMA,
        pltpu.SemaphoreType.DMA,
    ],
    compiler_params=pltpu.CompilerParams(needs_layout_passes=False),
)
def _sc_gather_scatter(h_hbm, hb_hbm, src0_hbm, dst0_hbm, src1_hbm, dst1_hbm,
                       zrows_hbm, zdeg_hbm,
                       agg_out, deg_out,
                       src_i0, src_i1, src_i2, dst_i0, dst_i1, dst_i2,
                       rows_v0, rows_v1, deg_l, agg_sh,
                       gsem0, gsem1, ssem0, ssem1, isem):
    c = lax.axis_index("c")
    s = lax.axis_index("s")
    src_i = (src_i0, src_i1, src_i2)
    dst_i = (dst_i0, dst_i1, dst_i2)
    rows_v = (rows_v0, rows_v1)
    gsem = (gsem0, gsem1)
    ssem = (ssem0, ssem1)

    @pl.when(s < N // RPT)
    def _():
        pltpu.sync_copy(zrows_hbm, agg_sh.at[pl.ds(s * RPT, RPT)])

    pltpu.sync_copy(zdeg_hbm, deg_l)
    plsc.subcore_barrier()

    @pl.when(c == 0)
    def _():
        _edge_pipeline(h_hbm, src0_hbm, dst0_hbm, CH0, s,
                       src_i, dst_i, rows_v, deg_l, agg_sh, gsem, ssem, isem)

    @pl.when(c == 1)
    def _():
        _edge_pipeline(hb_hbm, src1_hbm, dst1_hbm, CH1, s,
                       src_i, dst_i, rows_v, deg_l, agg_sh, gsem, ssem, isem)

    plsc.subcore_barrier()

    @pl.when(s < N // RPT)
    def _():
        pltpu.sync_copy(agg_sh.at[pl.ds(s * RPT, RPT)],
                        agg_out.at[c, pl.ds(s * RPT, RPT)])

    pltpu.sync_copy(deg_l, deg_out.at[c * NS + s])


def _proj_body(x_ref, w_ref, b_ref, o_ref, o2_ref):
    t = jnp.dot(x_ref[...], w_ref[...],
                preferred_element_type=jnp.float32) + b_ref[...]
    o_ref[...] = t
    o2_ref[...] = t


def _sage_block(agg_ref, deg_ref, h_ref, wl_ref, wr_ref, bl_ref, g_ref, be_ref):
    a = agg_ref[0] + agg_ref[1]
    deg = jnp.maximum(jnp.sum(deg_ref[0], axis=0), 1.0)
    a = a / deg[:, None]
    t = (jnp.dot(a, wl_ref[...], preferred_element_type=jnp.float32)
         + jnp.dot(h_ref[...], wr_ref[...], preferred_element_type=jnp.float32)
         + bl_ref[...])
    t = jnp.where(t > 0, t, jnp.exp(jnp.minimum(t, 0.0)) - 1.0)
    mu = jnp.mean(t, axis=-1, keepdims=True)
    var = jnp.mean((t - mu) ** 2, axis=-1, keepdims=True)
    return (t - mu) / jnp.sqrt(var + EPS) * g_ref[...] + be_ref[...]


def _combine_body(agg_ref, deg_ref, h_ref, wl_ref, wr_ref, bl_ref, g_ref,
                  be_ref, o_ref, o2_ref):
    t = _sage_block(agg_ref, deg_ref, h_ref, wl_ref, wr_ref, bl_ref,
                    g_ref, be_ref)
    o_ref[...] = t
    o2_ref[...] = t


def _combine_pool_body(agg_ref, deg_ref, h_ref, wl_ref, wr_ref, bl_ref, g_ref,
                       be_ref, batch_ref, wmt_ref, bmt_ref, o_ref,
                       sum_acc, cnt_acc):
    i = pl.program_id(0)

    @pl.when(i == 0)
    def _():
        sum_acc[...] = jnp.zeros_like(sum_acc)
        cnt_acc[...] = jnp.zeros_like(cnt_acc)

    hh = _sage_block(agg_ref, deg_ref, h_ref, wl_ref, wr_ref, bl_ref, g_ref,
                     be_ref)
    gi = lax.broadcasted_iota(jnp.int32, (G, RB), 0)
    oh = (batch_ref[0, 0, :][None, :] == gi).astype(jnp.float32)
    sum_acc[...] += jnp.dot(oh, hh, preferred_element_type=jnp.float32)
    cnt_acc[...] += jnp.broadcast_to(jnp.sum(oh, axis=1, keepdims=True), (G, D))

    @pl.when(i == NRB - 1)
    def _():
        emb = sum_acc[...] / jnp.maximum(cnt_acc[...], 1.0)
        o_ref[...] = jnp.dot(emb, wmt_ref[...],
                             preferred_element_type=jnp.float32) + bmt_ref[...]


_ROW_SPECS = [
    pl.BlockSpec((2, RB, D), lambda i: (0, i, 0)),
    pl.BlockSpec((1, NC * NS, RB), lambda i: (i, 0, 0)),
    pl.BlockSpec((RB, D), lambda i: (i, 0)),
    pl.BlockSpec((D, D), lambda i: (0, 0)),
    pl.BlockSpec((D, D), lambda i: (0, 0)),
    pl.BlockSpec((1, D), lambda i: (0, 0)),
    pl.BlockSpec((1, D), lambda i: (0, 0)),
    pl.BlockSpec((1, D), lambda i: (0, 0)),
]

_proj = pl.pallas_call(
    _proj_body,
    grid=(NRB,),
    in_specs=[
        pl.BlockSpec((RB, D), lambda i: (i, 0)),
        pl.BlockSpec((D, D), lambda i: (0, 0)),
        pl.BlockSpec((1, D), lambda i: (0, 0)),
    ],
    out_specs=[pl.BlockSpec((RB, D), lambda i: (i, 0)),
               pl.BlockSpec((RB, D), lambda i: (i, 0))],
    out_shape=[jax.ShapeDtypeStruct((N, D), jnp.float32),
               jax.ShapeDtypeStruct((N, D), jnp.float32)],
)

_combine = pl.pallas_call(
    _combine_body,
    grid=(NRB,),
    in_specs=_ROW_SPECS,
    out_specs=[pl.BlockSpec((RB, D), lambda i: (i, 0)),
               pl.BlockSpec((RB, D), lambda i: (i, 0))],
    out_shape=[jax.ShapeDtypeStruct((N, D), jnp.float32),
               jax.ShapeDtypeStruct((N, D), jnp.float32)],
)

_combine_pool = pl.pallas_call(
    _combine_pool_body,
    grid=(NRB,),
    in_specs=_ROW_SPECS + [
        pl.BlockSpec((1, 1, RB), lambda i: (i, 0, 0)),
        pl.BlockSpec((D, D), lambda i: (0, 0)),
        pl.BlockSpec((1, D), lambda i: (0, 0)),
    ],
    out_specs=pl.BlockSpec((G, D), lambda i: (0, 0)),
    out_shape=jax.ShapeDtypeStruct((G, D), jnp.float32),
    scratch_shapes=[
        pltpu.VMEM((G, D), jnp.float32),
        pltpu.VMEM((G, D), jnp.float32),
    ],
)


def kernel(x_operator, edge_index_calledby, batch_operator, W_op, b_op,
           W_l, b_l, W_r, gamma, beta, W_mem, b_mem, W_time, b_time):
    src = edge_index_calledby[0].astype(jnp.int32)
    dst = edge_index_calledby[1].astype(jnp.int32)
    pad = E_PAD - E
    e0 = NS * CH0 * K
    src_p = jnp.concatenate([src, jnp.zeros((pad,), jnp.int32)])
    src0 = src_p[:e0].reshape(NS, CH0, K)
    src1 = src_p[e0:].reshape(NS, CH1, K)
    dst_p = jnp.concatenate([dst, jnp.full((pad,), N, jnp.int32)])
    dst0 = dst_p[:e0].reshape(NS, CH0, K)
    dst1 = dst_p[e0:].reshape(NS, CH1, K)
    batch3 = batch_operator.astype(jnp.int32).reshape(NRB, 1, RB)

    zrows = jnp.zeros((RPT, D), jnp.float32)
    zdeg = jnp.zeros((NPAD,), jnp.float32)

    b_op2 = b_op.reshape(1, D)
    bl2 = b_l.reshape(1, D)
    g2 = gamma.reshape(1, D)
    be2 = beta.reshape(1, D)
    wmt = jnp.zeros((D, D), jnp.float32)
    wmt = wmt.at[:, 0].set(W_mem[:, 0]).at[:, 1].set(W_time[:, 0])
    bmt = jnp.zeros((1, D), jnp.float32)
    bmt = bmt.at[0, 0].set(b_mem[0]).at[0, 1].set(b_time[0])

    h0, h0b = _proj(x_operator, W_op, b_op2)
    aggp1, degp = _sc_gather_scatter(h0, h0b, src0, dst0, src1, dst1,
                                     zrows, zdeg)
    deg3 = degp[:, :N].reshape(NC * NS, NRB, RB).transpose(1, 0, 2)
    h1, h1b = _combine(aggp1, deg3, h0, W_l, W_r, bl2, g2, be2)
    aggp2, _ = _sc_gather_scatter(h1, h1b, src0, dst0, src1, dst1,
                                  zrows, zdeg)
    out = _combine_pool(aggp2, deg3, h1, W_l, W_r, bl2, g2, be2,
                        batch3, wmt, bmt)
    return out[:, 0], out[:, 1]


# speedup vs baseline: 2.5008x; 1.0117x over previous
import functools

import jax
import jax.numpy as jnp
from jax import lax
from jax.experimental import pallas as pl
from jax.experimental.pallas import tpu as pltpu
from jax.experimental.pallas import tpu_sc as plsc

N = 10000
D = 128
E = 320000
G = 256
NC = 2
NS = 16
K = 112
CH = 90
CH0 = 126
CH1 = 2 * CH - CH0
E_PAD = NC * NS * CH * K
NPAD = N + 8
RPT = 1000
RB = 1000
NRB = N // RB
EPS = 1e-5


_SC_MESH = plsc.VectorSubcoreMesh(core_axis_name="c", subcore_axis_name="s")


def _edge_pipeline(h_hbm, src_hbm, dst_hbm, n_chunks, s,
                   src_i, dst_i, rows_v, deg_l, agg_sh, gsem, ssem, isem):
    ones16 = jnp.ones((16,), jnp.float32)
    nj = n_chunks // 6

    def gather_wait(t, b):
        pltpu.make_async_copy(h_hbm.at[src_i[t]], rows_v[b], gsem[b]).wait()

    def scatter_wait(t, b):
        pltpu.make_async_copy(rows_v[b], agg_sh.at[dst_i[t]], ssem[b]).wait()

    pltpu.sync_copy(src_hbm.at[s, 0], src_i[0])
    pltpu.sync_copy(dst_hbm.at[s, 0], dst_i[0])
    pltpu.async_copy(h_hbm.at[src_i[0]], rows_v[0], gsem[0])

    def body(jj, carry):
        s_prev = None
        g_cur = None
        for u in range(6):
            j = jj * 6 + u
            b, bp = u % 2, 1 - u % 2
            t, tn = u % 3, (u + 1) % 3

            if u < 5:
                ia = pltpu.async_copy(src_hbm.at[s, j + 1], src_i[tn], isem)
                ib = pltpu.async_copy(dst_hbm.at[s, j + 1], dst_i[tn], isem)

            for k in range(K // 16):
                d16 = dst_i[t][pl.ds(k * 16, 16)]
                plsc.addupdate_scatter(deg_l, [d16], ones16)

            if g_cur is None:
                gather_wait(t, b)
            else:
                g_cur.wait()
            s_cur = pltpu.async_copy(rows_v[b], agg_sh.at[dst_i[t]], ssem[b],
                                     add=True)

            if s_prev is None:
                @pl.when(jj > 0)
                def _():
                    scatter_wait(2, 1)
            else:
                s_prev.wait()
            s_prev = s_cur

            if u < 5:
                ia.wait()
                ib.wait()
                g_cur = pltpu.async_copy(h_hbm.at[src_i[tn]], rows_v[bp],
                                         gsem[bp])
            else:
                @pl.when(jj < nj - 1)
                def _():
                    pltpu.async_copy(src_hbm.at[s, j + 1], src_i[tn],
                                     isem).wait()
                    pltpu.async_copy(dst_hbm.at[s, j + 1], dst_i[tn],
                                     isem).wait()
                    pltpu.async_copy(h_hbm.at[src_i[tn]], rows_v[bp], gsem[bp])
                g_cur = None
        return carry

    lax.fori_loop(0, nj, body, 0)
    scatter_wait(2, 1)


@functools.partial(
    pl.kernel,
    out_type=[
        jax.ShapeDtypeStruct((NC, N, D), jnp.float32),
        jax.ShapeDtypeStruct((NC * NS, NPAD), jnp.float32),
    ],
    mesh=_SC_MESH,
    scratch_types=[
        pltpu.VMEM((K,), jnp.int32),
        pltpu.VMEM((K,), jnp.int32),
        pltpu.VMEM((K,), jnp.int32),
        pltpu.VMEM((K,), jnp.int32),
        pltpu.VMEM((K,), jnp.int32),
        pltpu.VMEM((K,), jnp.int32),
        pltpu.VMEM((K, D), jnp.float32),
        pltpu.VMEM((K, D), jnp.float32),
        pltpu.VMEM((NPAD,), jnp.float32),
        pltpu.VMEM_SHARED((NPAD, D), jnp.float32),
        pltpu.SemaphoreType.DMA,
        pltpu.SemaphoreType.DMA,
        pltpu.SemaphoreType.DMA,
        pltpu.SemaphoreType.DMA,
        pltpu.SemaphoreType.DMA,
    ],
    compiler_params=pltpu.CompilerParams(needs_layout_passes=False),
)
def _sc_gather_scatter(h_hbm, src0_hbm, dst0_hbm, src1_hbm, dst1_hbm,
                       zrows_hbm, zdeg_hbm,
                       agg_out, deg_out,
                       src_i0, src_i1, src_i2, dst_i0, dst_i1, dst_i2,
                       rows_v0, rows_v1, deg_l, agg_sh,
                       gsem0, gsem1, ssem0, ssem1, isem):
    c = lax.axis_index("c")
    s = lax.axis_index("s")
    src_i = (src_i0, src_i1, src_i2)
    dst_i = (dst_i0, dst_i1, dst_i2)
    rows_v = (rows_v0, rows_v1)
    gsem = (gsem0, gsem1)
    ssem = (ssem0, ssem1)

    @pl.when(s < N // RPT)
    def _():
        pltpu.sync_copy(zrows_hbm, agg_sh.at[pl.ds(s * RPT, RPT)])

    pltpu.sync_copy(zdeg_hbm, deg_l)
    plsc.subcore_barrier()

    @pl.when(c == 0)
    def _():
        _edge_pipeline(h_hbm, src0_hbm, dst0_hbm, CH0, s,
                       src_i, dst_i, rows_v, deg_l, agg_sh, gsem, ssem, isem)

    @pl.when(c == 1)
    def _():
        _edge_pipeline(h_hbm, src1_hbm, dst1_hbm, CH1, s,
                       src_i, dst_i, rows_v, deg_l, agg_sh, gsem, ssem, isem)

    plsc.subcore_barrier()

    @pl.when(s < N // RPT)
    def _():
        pltpu.sync_copy(agg_sh.at[pl.ds(s * RPT, RPT)],
                        agg_out.at[c, pl.ds(s * RPT, RPT)])

    pltpu.sync_copy(deg_l, deg_out.at[c * NS + s])


def _proj_body(x_ref, w_ref, b_ref, o_ref):
    o_ref[...] = jnp.dot(x_ref[...], w_ref[...],
                         preferred_element_type=jnp.float32) + b_ref[...]


def _sage_block(agg_ref, deg_ref, h_ref, wl_ref, wr_ref, bl_ref, g_ref, be_ref):
    a = agg_ref[0] + agg_ref[1]
    deg = jnp.maximum(jnp.sum(deg_ref[0], axis=0), 1.0)
    a = a / deg[:, None]
    t = (jnp.dot(a, wl_ref[...], preferred_element_type=jnp.float32)
         + jnp.dot(h_ref[...], wr_ref[...], preferred_element_type=jnp.float32)
         + bl_ref[...])
    t = jnp.where(t > 0, t, jnp.exp(jnp.minimum(t, 0.0)) - 1.0)
    mu = jnp.mean(t, axis=-1, keepdims=True)
    var = jnp.mean((t - mu) ** 2, axis=-1, keepdims=True)
    return (t - mu) / jnp.sqrt(var + EPS) * g_ref[...] + be_ref[...]


def _combine_body(agg_ref, deg_ref, h_ref, wl_ref, wr_ref, bl_ref, g_ref,
                  be_ref, o_ref):
    o_ref[...] = _sage_block(agg_ref, deg_ref, h_ref, wl_ref, wr_ref, bl_ref,
                             g_ref, be_ref)


def _combine_pool_body(agg_ref, deg_ref, h_ref, wl_ref, wr_ref, bl_ref, g_ref,
                       be_ref, batch_ref, wmt_ref, bmt_ref, o_ref,
                       sum_acc, cnt_acc):
    i = pl.program_id(0)

    @pl.when(i == 0)
    def _():
        sum_acc[...] = jnp.zeros_like(sum_acc)
        cnt_acc[...] = jnp.zeros_like(cnt_acc)

    hh = _sage_block(agg_ref, deg_ref, h_ref, wl_ref, wr_ref, bl_ref, g_ref,
                     be_ref)
    gi = lax.broadcasted_iota(jnp.int32, (G, RB), 0)
    oh = (batch_ref[0, 0, :][None, :] == gi).astype(jnp.float32)
    sum_acc[...] += jnp.dot(oh, hh, preferred_element_type=jnp.float32)
    cnt_acc[...] += jnp.broadcast_to(jnp.sum(oh, axis=1, keepdims=True), (G, D))

    @pl.when(i == NRB - 1)
    def _():
        emb = sum_acc[...] / jnp.maximum(cnt_acc[...], 1.0)
        o_ref[...] = jnp.dot(emb, wmt_ref[...],
                             preferred_element_type=jnp.float32) + bmt_ref[...]


_ROW_SPECS = [
    pl.BlockSpec((2, RB, D), lambda i: (0, i, 0)),
    pl.BlockSpec((1, NC * NS, RB), lambda i: (i, 0, 0)),
    pl.BlockSpec((RB, D), lambda i: (i, 0)),
    pl.BlockSpec((D, D), lambda i: (0, 0)),
    pl.BlockSpec((D, D), lambda i: (0, 0)),
    pl.BlockSpec((1, D), lambda i: (0, 0)),
    pl.BlockSpec((1, D), lambda i: (0, 0)),
    pl.BlockSpec((1, D), lambda i: (0, 0)),
]

_proj = pl.pallas_call(
    _proj_body,
    grid=(NRB,),
    in_specs=[
        pl.BlockSpec((RB, D), lambda i: (i, 0)),
        pl.BlockSpec((D, D), lambda i: (0, 0)),
        pl.BlockSpec((1, D), lambda i: (0, 0)),
    ],
    out_specs=pl.BlockSpec((RB, D), lambda i: (i, 0)),
    out_shape=jax.ShapeDtypeStruct((N, D), jnp.float32),
)

_combine = pl.pallas_call(
    _combine_body,
    grid=(NRB,),
    in_specs=_ROW_SPECS,
    out_specs=pl.BlockSpec((RB, D), lambda i: (i, 0)),
    out_shape=jax.ShapeDtypeStruct((N, D), jnp.float32),
)

_combine_pool = pl.pallas_call(
    _combine_pool_body,
    grid=(NRB,),
    in_specs=_ROW_SPECS + [
        pl.BlockSpec((1, 1, RB), lambda i: (i, 0, 0)),
        pl.BlockSpec((D, D), lambda i: (0, 0)),
        pl.BlockSpec((1, D), lambda i: (0, 0)),
    ],
    out_specs=pl.BlockSpec((G, D), lambda i: (0, 0)),
    out_shape=jax.ShapeDtypeStruct((G, D), jnp.float32),
    scratch_shapes=[
        pltpu.VMEM((G, D), jnp.float32),
        pltpu.VMEM((G, D), jnp.float32),
    ],
)


def kernel(x_operator, edge_index_calledby, batch_operator, W_op, b_op,
           W_l, b_l, W_r, gamma, beta, W_mem, b_mem, W_time, b_time):
    src = edge_index_calledby[0].astype(jnp.int32)
    dst = edge_index_calledby[1].astype(jnp.int32)
    pad = E_PAD - E
    e0 = NS * CH0 * K
    src_p = jnp.concatenate([src, jnp.zeros((pad,), jnp.int32)])
    src0 = src_p[:e0].reshape(NS, CH0, K)
    src1 = src_p[e0:].reshape(NS, CH1, K)
    dst_p = jnp.concatenate([dst, jnp.full((pad,), N, jnp.int32)])
    dst0 = dst_p[:e0].reshape(NS, CH0, K)
    dst1 = dst_p[e0:].reshape(NS, CH1, K)
    batch3 = batch_operator.astype(jnp.int32).reshape(NRB, 1, RB)

    zrows = jnp.zeros((RPT, D), jnp.float32)
    zdeg = jnp.zeros((NPAD,), jnp.float32)

    b_op2 = b_op.reshape(1, D)
    bl2 = b_l.reshape(1, D)
    g2 = gamma.reshape(1, D)
    be2 = beta.reshape(1, D)
    wmt = jnp.zeros((D, D), jnp.float32)
    wmt = wmt.at[:, 0].set(W_mem[:, 0]).at[:, 1].set(W_time[:, 0])
    bmt = jnp.zeros((1, D), jnp.float32)
    bmt = bmt.at[0, 0].set(b_mem[0]).at[0, 1].set(b_time[0])

    h0 = _proj(x_operator, W_op, b_op2)
    aggp1, degp = _sc_gather_scatter(h0, src0, dst0, src1, dst1, zrows, zdeg)
    deg3 = degp[:, :N].reshape(NC * NS, NRB, RB).transpose(1, 0, 2)
    h1 = _combine(aggp1, deg3, h0, W_l, W_r, bl2, g2, be2)
    aggp2, _ = _sc_gather_scatter(h1, src0, dst0, src1, dst1, zrows, zdeg)
    out = _combine_pool(aggp2, deg3, h1, W_l, W_r, bl2, g2, be2,
                        batch3, wmt, bmt)
    return out[:, 0], out[:, 1]


# speedup vs baseline: 2.5018x; 1.0004x over previous
import functools

import jax
import jax.numpy as jnp
from jax import lax
from jax.experimental import pallas as pl
from jax.experimental.pallas import tpu as pltpu
from jax.experimental.pallas import tpu_sc as plsc

N = 10000
D = 128
E = 320000
G = 256
NC = 2
NS = 16
K = 112
CH = 90
CH0 = 126
CH1 = 2 * CH - CH0
E_PAD = NC * NS * CH * K
NPAD = N + 8
RPT = 1000
RB = 2000
NRB = N // RB
EPS = 1e-5


_SC_MESH = plsc.VectorSubcoreMesh(core_axis_name="c", subcore_axis_name="s")


def _edge_pipeline(h_hbm, src_hbm, dst_hbm, n_chunks, s,
                   src_i, dst_i, rows_v, deg_l, agg_sh, gsem, ssem, isem):
    ones16 = jnp.ones((16,), jnp.float32)
    nj = n_chunks // 6

    def gather_wait(t, b):
        pltpu.make_async_copy(h_hbm.at[src_i[t]], rows_v[b], gsem[b]).wait()

    def scatter_wait(t, b):
        pltpu.make_async_copy(rows_v[b], agg_sh.at[dst_i[t]], ssem[b]).wait()

    pltpu.sync_copy(src_hbm.at[s, 0], src_i[0])
    pltpu.sync_copy(dst_hbm.at[s, 0], dst_i[0])
    pltpu.async_copy(h_hbm.at[src_i[0]], rows_v[0], gsem[0])

    def body(jj, carry):
        s_prev = None
        g_cur = None
        for u in range(6):
            j = jj * 6 + u
            b, bp = u % 2, 1 - u % 2
            t, tn = u % 3, (u + 1) % 3

            if u < 5:
                ia = pltpu.async_copy(src_hbm.at[s, j + 1], src_i[tn], isem)
                ib = pltpu.async_copy(dst_hbm.at[s, j + 1], dst_i[tn], isem)

            for k in range(K // 16):
                d16 = dst_i[t][pl.ds(k * 16, 16)]
                plsc.addupdate_scatter(deg_l, [d16], ones16)

            if g_cur is None:
                gather_wait(t, b)
            else:
                g_cur.wait()
            s_cur = pltpu.async_copy(rows_v[b], agg_sh.at[dst_i[t]], ssem[b],
                                     add=True)

            if s_prev is None:
                @pl.when(jj > 0)
                def _():
                    scatter_wait(2, 1)
            else:
                s_prev.wait()
            s_prev = s_cur

            if u < 5:
                ia.wait()
                ib.wait()
                g_cur = pltpu.async_copy(h_hbm.at[src_i[tn]], rows_v[bp],
                                         gsem[bp])
            else:
                @pl.when(jj < nj - 1)
                def _():
                    pltpu.async_copy(src_hbm.at[s, j + 1], src_i[tn],
                                     isem).wait()
                    pltpu.async_copy(dst_hbm.at[s, j + 1], dst_i[tn],
                                     isem).wait()
                    pltpu.async_copy(h_hbm.at[src_i[tn]], rows_v[bp], gsem[bp])
                g_cur = None
        return carry

    lax.fori_loop(0, nj, body, 0)
    scatter_wait(2, 1)


@functools.partial(
    pl.kernel,
    out_type=[
        jax.ShapeDtypeStruct((NC, N, D), jnp.float32),
        jax.ShapeDtypeStruct((NC * NS, NPAD), jnp.float32),
    ],
    mesh=_SC_MESH,
    scratch_types=[
        pltpu.VMEM((K,), jnp.int32),
        pltpu.VMEM((K,), jnp.int32),
        pltpu.VMEM((K,), jnp.int32),
        pltpu.VMEM((K,), jnp.int32),
        pltpu.VMEM((K,), jnp.int32),
        pltpu.VMEM((K,), jnp.int32),
        pltpu.VMEM((K, D), jnp.float32),
        pltpu.VMEM((K, D), jnp.float32),
        pltpu.VMEM((NPAD,), jnp.float32),
        pltpu.VMEM_SHARED((NPAD, D), jnp.float32),
        pltpu.SemaphoreType.DMA,
        pltpu.SemaphoreType.DMA,
        pltpu.SemaphoreType.DMA,
        pltpu.SemaphoreType.DMA,
        pltpu.SemaphoreType.DMA,
    ],
    compiler_params=pltpu.CompilerParams(needs_layout_passes=False),
)
def _sc_gather_scatter(h_hbm, src0_hbm, dst0_hbm, src1_hbm, dst1_hbm,
                       zrows_hbm, zdeg_hbm,
                       agg_out, deg_out,
                       src_i0, src_i1, src_i2, dst_i0, dst_i1, dst_i2,
                       rows_v0, rows_v1, deg_l, agg_sh,
                       gsem0, gsem1, ssem0, ssem1, isem):
    c = lax.axis_index("c")
    s = lax.axis_index("s")
    src_i = (src_i0, src_i1, src_i2)
    dst_i = (dst_i0, dst_i1, dst_i2)
    rows_v = (rows_v0, rows_v1)
    gsem = (gsem0, gsem1)
    ssem = (ssem0, ssem1)

    @pl.when(s < N // RPT)
    def _():
        pltpu.sync_copy(zrows_hbm, agg_sh.at[pl.ds(s * RPT, RPT)])

    pltpu.sync_copy(zdeg_hbm, deg_l)
    plsc.subcore_barrier()

    @pl.when(c == 0)
    def _():
        _edge_pipeline(h_hbm, src0_hbm, dst0_hbm, CH0, s,
                       src_i, dst_i, rows_v, deg_l, agg_sh, gsem, ssem, isem)

    @pl.when(c == 1)
    def _():
        _edge_pipeline(h_hbm, src1_hbm, dst1_hbm, CH1, s,
                       src_i, dst_i, rows_v, deg_l, agg_sh, gsem, ssem, isem)

    plsc.subcore_barrier()

    @pl.when(s < N // RPT)
    def _():
        pltpu.sync_copy(agg_sh.at[pl.ds(s * RPT, RPT)],
                        agg_out.at[c, pl.ds(s * RPT, RPT)])

    pltpu.sync_copy(deg_l, deg_out.at[c * NS + s])


def _proj_body(x_ref, w_ref, b_ref, o_ref):
    o_ref[...] = jnp.dot(x_ref[...], w_ref[...],
                         preferred_element_type=jnp.float32) + b_ref[...]


def _sage_block(agg_ref, deg_ref, h_ref, wl_ref, wr_ref, bl_ref, g_ref, be_ref):
    a = agg_ref[0] + agg_ref[1]
    deg = jnp.maximum(jnp.sum(deg_ref[0], axis=0), 1.0)
    a = a / deg[:, None]
    t = (jnp.dot(a, wl_ref[...], preferred_element_type=jnp.float32)
         + jnp.dot(h_ref[...], wr_ref[...], preferred_element_type=jnp.float32)
         + bl_ref[...])
    t = jnp.where(t > 0, t, jnp.exp(jnp.minimum(t, 0.0)) - 1.0)
    mu = jnp.mean(t, axis=-1, keepdims=True)
    var = jnp.mean((t - mu) ** 2, axis=-1, keepdims=True)
    return (t - mu) / jnp.sqrt(var + EPS) * g_ref[...] + be_ref[...]


def _combine_body(agg_ref, deg_ref, h_ref, wl_ref, wr_ref, bl_ref, g_ref,
                  be_ref, o_ref):
    o_ref[...] = _sage_block(agg_ref, deg_ref, h_ref, wl_ref, wr_ref, bl_ref,
                             g_ref, be_ref)


def _combine_pool_body(agg_ref, deg_ref, h_ref, wl_ref, wr_ref, bl_ref, g_ref,
                       be_ref, batch_ref, wmt_ref, bmt_ref, o_ref,
                       sum_acc, cnt_acc):
    i = pl.program_id(0)

    @pl.when(i == 0)
    def _():
        sum_acc[...] = jnp.zeros_like(sum_acc)
        cnt_acc[...] = jnp.zeros_like(cnt_acc)

    hh = _sage_block(agg_ref, deg_ref, h_ref, wl_ref, wr_ref, bl_ref, g_ref,
                     be_ref)
    gi = lax.broadcasted_iota(jnp.int32, (G, RB), 0)
    oh = (batch_ref[0, 0, :][None, :] == gi).astype(jnp.float32)
    sum_acc[...] += jnp.dot(oh, hh, preferred_element_type=jnp.float32)
    cnt_acc[...] += jnp.broadcast_to(jnp.sum(oh, axis=1, keepdims=True), (G, D))

    @pl.when(i == NRB - 1)
    def _():
        emb = sum_acc[...] / jnp.maximum(cnt_acc[...], 1.0)
        o_ref[...] = jnp.dot(emb, wmt_ref[...],
                             preferred_element_type=jnp.float32) + bmt_ref[...]


_ROW_SPECS = [
    pl.BlockSpec((2, RB, D), lambda i: (0, i, 0)),
    pl.BlockSpec((1, NC * NS, RB), lambda i: (i, 0, 0)),
    pl.BlockSpec((RB, D), lambda i: (i, 0)),
    pl.BlockSpec((D, D), lambda i: (0, 0)),
    pl.BlockSpec((D, D), lambda i: (0, 0)),
    pl.BlockSpec((1, D), lambda i: (0, 0)),
    pl.BlockSpec((1, D), lambda i: (0, 0)),
    pl.BlockSpec((1, D), lambda i: (0, 0)),
]

_proj = pl.pallas_call(
    _proj_body,
    grid=(NRB,),
    in_specs=[
        pl.BlockSpec((RB, D), lambda i: (i, 0)),
        pl.BlockSpec((D, D), lambda i: (0, 0)),
        pl.BlockSpec((1, D), lambda i: (0, 0)),
    ],
    out_specs=pl.BlockSpec((RB, D), lambda i: (i, 0)),
    out_shape=jax.ShapeDtypeStruct((N, D), jnp.float32),
)

_combine = pl.pallas_call(
    _combine_body,
    grid=(NRB,),
    in_specs=_ROW_SPECS,
    out_specs=pl.BlockSpec((RB, D), lambda i: (i, 0)),
    out_shape=jax.ShapeDtypeStruct((N, D), jnp.float32),
)

_combine_pool = pl.pallas_call(
    _combine_pool_body,
    grid=(NRB,),
    in_specs=_ROW_SPECS + [
        pl.BlockSpec((1, 1, RB), lambda i: (i, 0, 0)),
        pl.BlockSpec((D, D), lambda i: (0, 0)),
        pl.BlockSpec((1, D), lambda i: (0, 0)),
    ],
    out_specs=pl.BlockSpec((G, D), lambda i: (0, 0)),
    out_shape=jax.ShapeDtypeStruct((G, D), jnp.float32),
    scratch_shapes=[
        pltpu.VMEM((G, D), jnp.float32),
        pltpu.VMEM((G, D), jnp.float32),
    ],
)


def kernel(x_operator, edge_index_calledby, batch_operator, W_op, b_op,
           W_l, b_l, W_r, gamma, beta, W_mem, b_mem, W_time, b_time):
    src = edge_index_calledby[0].astype(jnp.int32)
    dst = edge_index_calledby[1].astype(jnp.int32)
    pad = E_PAD - E
    e0 = NS * CH0 * K
    src_p = jnp.concatenate([src, jnp.zeros((pad,), jnp.int32)])
    src0 = src_p[:e0].reshape(NS, CH0, K)
    src1 = src_p[e0:].reshape(NS, CH1, K)
    dst_p = jnp.concatenate([dst, jnp.full((pad,), N, jnp.int32)])
    dst0 = dst_p[:e0].reshape(NS, CH0, K)
    dst1 = dst_p[e0:].reshape(NS, CH1, K)
    batch3 = batch_operator.astype(jnp.int32).reshape(NRB, 1, RB)

    zrows = jnp.zeros((RPT, D), jnp.float32)
    zdeg = jnp.zeros((NPAD,), jnp.float32)

    b_op2 = b_op.reshape(1, D)
    bl2 = b_l.reshape(1, D)
    g2 = gamma.reshape(1, D)
    be2 = beta.reshape(1, D)
    wmt = jnp.zeros((D, D), jnp.float32)
    wmt = wmt.at[:, 0].set(W_mem[:, 0]).at[:, 1].set(W_time[:, 0])
    bmt = jnp.zeros((1, D), jnp.float32)
    bmt = bmt.at[0, 0].set(b_mem[0]).at[0, 1].set(b_time[0])

    h0 = _proj(x_operator, W_op, b_op2)
    aggp1, degp = _sc_gather_scatter(h0, src0, dst0, src1, dst1, zrows, zdeg)
    deg3 = degp[:, :N].reshape(NC * NS, NRB, RB).transpose(1, 0, 2)
    h1 = _combine(aggp1, deg3, h0, W_l, W_r, bl2, g2, be2)
    aggp2, _ = _sc_gather_scatter(h1, src0, dst0, src1, dst1, zrows, zdeg)
    out = _combine_pool(aggp2, deg3, h1, W_l, W_r, bl2, g2, be2,
                        batch3, wmt, bmt)
    return out[:, 0], out[:, 1]
